# baseline (speedup 1.0000x reference)
import numpy as np
import jax
import jax.numpy as jnp
from jax.experimental import pallas as pl
from jax.experimental.pallas import tpu as pltpu

B = 64
V = 100000
CBLK = 2048
NBLK = (V + CBLK - 1) // CBLK

_TINY = np.float32(np.finfo(np.float32).tiny)
_KS1 = np.uint32(42)
_KS2 = np.uint32(42 ^ 0x1BD11BDA)


def _rotl(x, d):
    return (x << np.uint32(d)) | (x >> np.uint32(32 - d))


def _round4(x0, x1, rots):
    for r in rots:
        x0 = x0 + x1
        x1 = _rotl(x1, r)
        x1 = x0 ^ x1
    return x0, x1


def _threefry_bits(flat):
    A = (13, 15, 26, 6)
    Br = (17, 29, 16, 24)
    x1 = flat + _KS1
    x0 = x1
    x1 = x0 ^ _rotl(x1, A[0])
    x0, x1 = _round4(x0, x1, A[1:])
    x0, x1 = x0 + _KS1, x1 + np.uint32(_KS2 + np.uint32(1))
    x0, x1 = _round4(x0, x1, Br)
    x0, x1 = x0 + _KS2, x1 + np.uint32(2)
    x0, x1 = _round4(x0, x1, A)
    x1 = x1 + np.uint32(_KS1 + np.uint32(3))
    x0, x1 = _round4(x0, x1, Br)
    x0, x1 = x0 + _KS1, x1 + np.uint32(_KS2 + np.uint32(4))
    x0, x1 = _round4(x0, x1, A)
    x0, x1 = x0 + _KS2, x1 + np.uint32(5)
    return x0 ^ x1


def _body(logits_ref, mask_ref, adj_ref, ids_ref, maxv_ref, argm_ref, flat_ref):
    j = pl.program_id(0)
    adj = logits_ref[...] + mask_ref[...]
    adj_ref[...] = adj

    @pl.when(j == 0)
    def _():
        lane0 = jax.lax.broadcasted_iota(jnp.uint32, (B, CBLK), 1)
        row = jax.lax.broadcasted_iota(jnp.uint32, (B, CBLK), 0)
        flat_ref[...] = row * np.uint32(V) + lane0

    @pl.when(j > 0)
    def _():
        flat_ref[...] = flat_ref[...] + np.uint32(CBLK)

    bits = _threefry_bits(flat_ref[...])

    fbits = (bits >> np.uint32(9)) | np.uint32(0x3F800000)
    fl = jax.lax.bitcast_convert_type(fbits, jnp.float32) - np.float32(1.0)
    u = fl + _TINY
    pert = -jnp.log(-jnp.log(u)) + adj

    lane = jax.lax.broadcasted_iota(jnp.int32, (B, CBLK), 1)
    pert = jnp.where(lane < V - j * CBLK, pert, -jnp.inf)
    lmax = jnp.max(pert, axis=1, keepdims=True)
    cand = jnp.where(pert == lmax, lane, CBLK)
    larg = jnp.min(cand, axis=1, keepdims=True) + j * CBLK

    @pl.when(j == 0)
    def _():
        maxv_ref[...] = lmax
        argm_ref[...] = larg

    @pl.when(j > 0)
    def _():
        prev = maxv_ref[...]
        better = lmax > prev
        maxv_ref[...] = jnp.where(better, lmax, prev)
        argm_ref[...] = jnp.where(better, larg, argm_ref[...])

    @pl.when(j == NBLK - 1)
    def _():
        ids_ref[...] = argm_ref[...]


@jax.jit
def _run(predicted_logits, mask2d):
    adj, ids = pl.pallas_call(
        _body,
        grid=(NBLK,),
        in_specs=[
            pl.BlockSpec((B, CBLK), lambda j: (0, j)),
            pl.BlockSpec((1, CBLK), lambda j: (0, j)),
        ],
        out_specs=[
            pl.BlockSpec((B, CBLK), lambda j: (0, j)),
            pl.BlockSpec((B, 1), lambda j: (0, 0)),
        ],
        out_shape=[
            jax.ShapeDtypeStruct((B, V), jnp.float32),
            jax.ShapeDtypeStruct((B, 1), jnp.int32),
        ],
        scratch_shapes=[
            pltpu.VMEM((B, 1), jnp.float32),
            pltpu.VMEM((B, 1), jnp.int32),
            pltpu.VMEM((B, CBLK), jnp.uint32),
        ],
    )(predicted_logits, mask2d)
    return ids.reshape(B), adj


def kernel(predicted_logits, prediction_mask):
    ids, adj = _run(predicted_logits, prediction_mask.reshape(1, V))
    return (ids, adj)


# speedup vs baseline: 1.0534x; 1.0534x over previous
import numpy as np
import jax
import jax.numpy as jnp
from jax.experimental import pallas as pl
from jax.experimental.pallas import tpu as pltpu

B = 64
V = 100000
CBLK = 2048
NBLK = (V + CBLK - 1) // CBLK

_TINY = np.float32(np.finfo(np.float32).tiny)
_KS1 = np.uint32(42)
_KS2 = np.uint32(42 ^ 0x1BD11BDA)


def _rotl(x, d):
    return (x << np.uint32(d)) | (x >> np.uint32(32 - d))


def _round4(x0, x1, rots):
    for r in rots:
        x0 = x0 + x1
        x1 = _rotl(x1, r)
        x1 = x0 ^ x1
    return x0, x1


def _threefry_bits(flat):
    A = (13, 15, 26, 6)
    Br = (17, 29, 16, 24)
    x1 = flat + _KS1
    x0 = x1
    x1 = x0 ^ _rotl(x1, A[0])
    x0, x1 = _round4(x0, x1, A[1:])
    x0, x1 = x0 + _KS1, x1 + np.uint32(_KS2 + np.uint32(1))
    x0, x1 = _round4(x0, x1, Br)
    x0, x1 = x0 + _KS2, x1 + np.uint32(2)
    x0, x1 = _round4(x0, x1, A)
    x1 = x1 + np.uint32(_KS1 + np.uint32(3))
    x0, x1 = _round4(x0, x1, Br)
    x0, x1 = x0 + _KS1, x1 + np.uint32(_KS2 + np.uint32(4))
    x0, x1 = _round4(x0, x1, A)
    x0, x1 = x0 + _KS2, x1 + np.uint32(5)
    return x0 ^ x1


def _body(logits_ref, mask_ref, adj_ref, ids_ref, maxv_ref, argm_ref):
    j = pl.program_id(0)
    adj = logits_ref[...] + mask_ref[...]
    adj_ref[...] = adj

    col = jax.lax.broadcasted_iota(jnp.int32, (B, CBLK), 1) + j * CBLK
    row = jax.lax.broadcasted_iota(jnp.int32, (B, CBLK), 0)
    flat = (row * V + col).astype(jnp.uint32)
    bits = _threefry_bits(flat)

    fbits = (bits >> np.uint32(9)) | np.uint32(0x3F800000)
    fl = jax.lax.bitcast_convert_type(fbits, jnp.float32) - np.float32(1.0)
    u = fl + _TINY
    pert = -jnp.log(-jnp.log(u)) + adj

    pert = jnp.where(col < V, pert, -jnp.inf)
    lmax = jnp.max(pert, axis=1, keepdims=True)
    cand = jnp.where(pert == lmax, col, V)
    larg = jnp.min(cand, axis=1, keepdims=True)

    prev = jnp.where(j == 0, -jnp.inf, maxv_ref[...])
    better = lmax > prev
    maxv_ref[...] = jnp.where(better, lmax, prev)
    argm = jnp.where(better, larg, argm_ref[...])
    argm_ref[...] = argm

    @pl.when(j == NBLK - 1)
    def _():
        ids_ref[...] = argm


@jax.jit
def _run(predicted_logits, mask2d):
    adj, ids = pl.pallas_call(
        _body,
        grid=(NBLK,),
        in_specs=[
            pl.BlockSpec((B, CBLK), lambda j: (0, j)),
            pl.BlockSpec((1, CBLK), lambda j: (0, j)),
        ],
        out_specs=[
            pl.BlockSpec((B, CBLK), lambda j: (0, j)),
            pl.BlockSpec((B, 1), lambda j: (0, 0)),
        ],
        out_shape=[
            jax.ShapeDtypeStruct((B, V), jnp.float32),
            jax.ShapeDtypeStruct((B, 1), jnp.int32),
        ],
        scratch_shapes=[
            pltpu.VMEM((B, 1), jnp.float32),
            pltpu.VMEM((B, 1), jnp.int32),
        ],
    )(predicted_logits, mask2d)
    return ids.reshape(B), adj


def kernel(predicted_logits, prediction_mask):
    ids, adj = _run(predicted_logits, prediction_mask.reshape(1, V))
    return (ids, adj)
